# baseline (speedup 1.0000x reference)
import functools

import jax
import jax.numpy as jnp
from jax import lax
from jax.experimental import pallas as pl
from jax.experimental.pallas import tpu as pltpu
from jax.experimental.pallas import tpu_sc as plsc

CODES = 1000000
WIDTH = 64
BATCH = 16384
FIELDS = 26
FPAD = 32

NC = 2
NS = 16
LANES = 16
NW = NC * NS
BP = BATCH * FPAD
BPW = BP // NW
CHUNK = 128
NCHUNK = BPW // CHUNK
NBUF = 4
NGROUP = NCHUNK // NBUF

_mesh = plsc.VectorSubcoreMesh(
    core_axis_name="c", subcore_axis_name="s", num_cores=NC, num_subcores=NS
)


@functools.partial(
    pl.kernel,
    out_type=jax.ShapeDtypeStruct((BP, WIDTH), jnp.float32),
    mesh=_mesh,
    compiler_params=pltpu.CompilerParams(use_tc_tiling_on_sc=False),
    scratch_types=[
        pltpu.VMEM((BPW,), jnp.float32),
        pltpu.VMEM((BPW,), jnp.int32),
        pltpu.VMEM((NBUF, CHUNK, WIDTH), jnp.float32),
        [pltpu.SemaphoreType.DMA] * NBUF,
    ],
)
def _embed_gather(x_hbm, tab_hbm, out_hbm, x_v, idx_v, rows_v, gsems):
    wid = lax.axis_index("s") * NC + lax.axis_index("c")
    base = wid * BPW

    pltpu.sync_copy(x_hbm.at[pl.ds(base, BPW)], x_v)

    def conv(i, carry):
        v = x_v[pl.ds(i * LANES, LANES)]
        v = jnp.where(v != v, jnp.float32(CODES), v)
        idx_v[pl.ds(i * LANES, LANES)] = v.astype(jnp.int32)
        return carry

    lax.fori_loop(0, BPW // LANES, conv, 0, unroll=4)

    def gather(j, b):
        return pltpu.make_async_copy(
            tab_hbm.at[idx_v.at[pl.ds(j * CHUNK, CHUNK)]], rows_v.at[b], gsems[b]
        )

    def write_out(j, b):
        pltpu.sync_copy(rows_v.at[b], out_hbm.at[pl.ds(base + j * CHUNK, CHUNK)])

    for b in range(NBUF):
        gather(b, b).start()

    def group(gi, carry):
        for b in range(NBUF):
            j = gi * NBUF + b
            gather(j, b).wait()
            write_out(j, b)
            gather(j + NBUF, b).start()
        return carry

    lax.fori_loop(0, NGROUP - 1, group, 0)

    for b in range(NBUF):
        j = (NGROUP - 1) * NBUF + b
        gather(j, b).wait()
        write_out(j, b)


_BB = 512


def _slice_body(in_ref, out_ref):
    out_ref[...] = in_ref[:, :FIELDS, :]


_unpad = pl.pallas_call(
    _slice_body,
    out_shape=jax.ShapeDtypeStruct((BATCH, FIELDS, WIDTH), jnp.float32),
    grid=(BATCH // _BB,),
    in_specs=[pl.BlockSpec((_BB, FPAD, WIDTH), lambda i: (i, 0, 0))],
    out_specs=pl.BlockSpec((_BB, FIELDS, WIDTH), lambda i: (i, 0, 0)),
)


def kernel(x, embed):
    xp = jnp.pad(x, ((0, 0), (0, FPAD - FIELDS)))
    rows = _embed_gather(xp.reshape(BP), embed)
    return _unpad(rows.reshape(BATCH, FPAD, WIDTH))


# speedup vs baseline: 2.4473x; 2.4473x over previous
import functools

import jax
import jax.numpy as jnp
from jax import lax
from jax.experimental import pallas as pl
from jax.experimental.pallas import tpu as pltpu
from jax.experimental.pallas import tpu_sc as plsc

CODES = 1000000
WIDTH = 64
BATCH = 16384
FIELDS = 26

NC = 2
NS = 16
NW = NC * NS
B = BATCH * FIELDS
BPW = B // NW
CHUNK = 128
NCHUNK = BPW // CHUNK
NBUF = 4
NGROUP = NCHUNK // NBUF

_mesh = plsc.VectorSubcoreMesh(
    core_axis_name="c", subcore_axis_name="s", num_cores=NC, num_subcores=NS
)


@functools.partial(
    pl.kernel,
    out_type=jax.ShapeDtypeStruct((B, WIDTH), jnp.float32),
    mesh=_mesh,
    compiler_params=pltpu.CompilerParams(use_tc_tiling_on_sc=False),
    scratch_types=[
        pltpu.VMEM((BPW,), jnp.int32),
        pltpu.VMEM((NBUF, CHUNK, WIDTH), jnp.float32),
        [pltpu.SemaphoreType.DMA] * NBUF,
    ],
)
def _embed_gather(idx_hbm, tab_hbm, out_hbm, idx_v, rows_v, gsems):
    wid = lax.axis_index("s") * NC + lax.axis_index("c")
    base = wid * BPW

    pltpu.sync_copy(idx_hbm.at[pl.ds(base, BPW)], idx_v)

    def gather(j, b):
        return pltpu.make_async_copy(
            tab_hbm.at[idx_v.at[pl.ds(j * CHUNK, CHUNK)]], rows_v.at[b], gsems[b]
        )

    def write_out(j, b):
        pltpu.sync_copy(rows_v.at[b], out_hbm.at[pl.ds(base + j * CHUNK, CHUNK)])

    for b in range(NBUF):
        gather(b, b).start()

    def group(gi, carry):
        for b in range(NBUF):
            j = gi * NBUF + b
            gather(j, b).wait()
            write_out(j, b)
            gather(j + NBUF, b).start()
        return carry

    lax.fori_loop(0, NGROUP - 1, group, 0)

    for b in range(NBUF):
        j = (NGROUP - 1) * NBUF + b
        gather(j, b).wait()
        write_out(j, b)


_BB = 512


def _tr_body(in_ref, out_ref):
    out_ref[...] = jnp.transpose(in_ref[...], (0, 2, 1))


_transpose = pl.pallas_call(
    _tr_body,
    out_shape=jax.ShapeDtypeStruct((FIELDS, WIDTH, BATCH), jnp.float32),
    grid=(FIELDS, BATCH // _BB),
    in_specs=[pl.BlockSpec((1, _BB, WIDTH), lambda f, i: (f, i, 0))],
    out_specs=pl.BlockSpec((1, WIDTH, _BB), lambda f, i: (f, 0, i)),
)


def kernel(x, embed):
    idx = jnp.where(jnp.isnan(x), jnp.float32(CODES), x).astype(jnp.int32)
    idx = idx.T.reshape(B)
    rows = _embed_gather(idx, embed)
    out = _transpose(rows.reshape(FIELDS, BATCH, WIDTH))
    return jnp.transpose(out, (2, 0, 1))


# speedup vs baseline: 4.5969x; 1.8784x over previous
import functools

import jax
import jax.numpy as jnp
from jax import lax
from jax.experimental import pallas as pl
from jax.experimental.pallas import tpu as pltpu
from jax.experimental.pallas import tpu_sc as plsc

CODES = 1000000
WIDTH = 64
BATCH = 16384
FIELDS = 26

NC = 2
NS = 16
NW = NC * NS
B = BATCH * FIELDS
BPW = B // NW
CHUNK = 128
NCHUNK = BPW // CHUNK
NBUF = 4
NGROUP = NCHUNK // NBUF

_mesh = plsc.VectorSubcoreMesh(
    core_axis_name="c", subcore_axis_name="s", num_cores=NC, num_subcores=NS
)


@functools.partial(
    pl.kernel,
    out_type=jax.ShapeDtypeStruct((B, WIDTH), jnp.float32),
    mesh=_mesh,
    compiler_params=pltpu.CompilerParams(use_tc_tiling_on_sc=False),
    scratch_types=[
        pltpu.VMEM((BPW,), jnp.int32),
        pltpu.VMEM((NBUF, CHUNK, WIDTH), jnp.float32),
        [pltpu.SemaphoreType.DMA] * NBUF,
    ],
)
def _embed_gather(idx_hbm, tab_hbm, out_hbm, idx_v, rows_v, gsems):
    wid = lax.axis_index("s") * NC + lax.axis_index("c")
    base = wid * BPW

    pltpu.sync_copy(idx_hbm.at[pl.ds(base, BPW)], idx_v)

    def gather(j, b):
        return pltpu.make_async_copy(
            tab_hbm.at[idx_v.at[pl.ds(j * CHUNK, CHUNK)]], rows_v.at[b], gsems[b]
        )

    def write_out(j, b):
        pltpu.sync_copy(rows_v.at[b], out_hbm.at[pl.ds(base + j * CHUNK, CHUNK)])

    for b in range(NBUF):
        gather(b, b).start()

    def group(gi, carry):
        for b in range(NBUF):
            j = gi * NBUF + b
            gather(j, b).wait()
            write_out(j, b)
            gather(j + NBUF, b).start()
        return carry

    lax.fori_loop(0, NGROUP - 1, group, 0)

    for b in range(NBUF):
        j = (NGROUP - 1) * NBUF + b
        gather(j, b).wait()
        write_out(j, b)


_BB = 4096


def _tr_body(in_ref, out_ref):
    out_ref[...] = jnp.transpose(in_ref[...], (0, 2, 1))


_transpose = pl.pallas_call(
    _tr_body,
    out_shape=jax.ShapeDtypeStruct((FIELDS, WIDTH, BATCH), jnp.float32),
    grid=(FIELDS, BATCH // _BB),
    in_specs=[pl.BlockSpec((1, _BB, WIDTH), lambda f, i: (f, i, 0))],
    out_specs=pl.BlockSpec((1, WIDTH, _BB), lambda f, i: (f, 0, i)),
)

_PB = 2048
_NP = 501760


def _pack_body(a_ref, b_ref, out_ref):
    out_ref[:, :WIDTH] = jnp.transpose(a_ref[...], (1, 0))
    out_ref[:, WIDTH:] = jnp.transpose(b_ref[...], (1, 0))


_pack = pl.pallas_call(
    _pack_body,
    out_shape=jax.ShapeDtypeStruct((_NP, 2 * WIDTH), jnp.float32),
    grid=(_NP // _PB,),
    in_specs=[
        pl.BlockSpec((WIDTH, _PB), lambda i: (0, i)),
        pl.BlockSpec(
            (WIDTH, _PB),
            lambda i: (0, jnp.minimum(i + _NP // _PB, CODES // _PB)),
        ),
    ],
    out_specs=pl.BlockSpec((_PB, 2 * WIDTH), lambda i: (i, 0)),
)


def kernel(x, embed):
    v = jnp.where(jnp.isnan(x), jnp.float32(CODES), x).astype(jnp.int32)
    u = jnp.where(v < _NP, 2 * v, 2 * (v - _NP) + 1)
    u = u.T.reshape(B)
    tab = _pack(jnp.transpose(embed), jnp.transpose(embed))
    rows = _embed_gather(u, tab.reshape(2 * _NP, WIDTH))
    out = _transpose(rows.reshape(FIELDS, BATCH, WIDTH))
    return jnp.transpose(out, (2, 0, 1))


# speedup vs baseline: 5.3328x; 1.1601x over previous
import functools

import jax
import jax.numpy as jnp
from jax import lax
from jax.experimental import pallas as pl
from jax.experimental.pallas import tpu as pltpu
from jax.experimental.pallas import tpu_sc as plsc

CODES = 1000000
WIDTH = 64
BATCH = 16384
FIELDS = 26

NC = 2
NS = 16
NW = NC * NS
B = BATCH * FIELDS
BPW = B // NW
CHUNK = 128
NCHUNK = BPW // CHUNK
NBUF = 4
NGROUP = NCHUNK // NBUF

_mesh = plsc.VectorSubcoreMesh(
    core_axis_name="c", subcore_axis_name="s", num_cores=NC, num_subcores=NS
)


@functools.partial(
    pl.kernel,
    out_type=jax.ShapeDtypeStruct((B, WIDTH), jnp.float32),
    mesh=_mesh,
    compiler_params=pltpu.CompilerParams(use_tc_tiling_on_sc=False),
    scratch_types=[
        pltpu.VMEM((BPW,), jnp.int32),
        pltpu.VMEM((NBUF, CHUNK, WIDTH), jnp.float32),
        [pltpu.SemaphoreType.DMA] * NBUF,
    ],
)
def _embed_gather(idx_hbm, tab_hbm, out_hbm, idx_v, rows_v, gsems):
    wid = lax.axis_index("s") * NC + lax.axis_index("c")
    base = wid * BPW

    pltpu.sync_copy(idx_hbm.at[pl.ds(base, BPW)], idx_v)

    def gather(j, b):
        return pltpu.make_async_copy(
            tab_hbm.at[idx_v.at[pl.ds(j * CHUNK, CHUNK)]], rows_v.at[b], gsems[b]
        )

    def write_out(j, b):
        pltpu.sync_copy(rows_v.at[b], out_hbm.at[pl.ds(base + j * CHUNK, CHUNK)])

    for b in range(NBUF):
        gather(b, b).start()

    def group(gi, carry):
        for b in range(NBUF):
            j = gi * NBUF + b
            gather(j, b).wait()
            write_out(j, b)
            gather(j + NBUF, b).start()
        return carry

    lax.fori_loop(0, NGROUP - 1, group, 0)

    for b in range(NBUF):
        j = (NGROUP - 1) * NBUF + b
        gather(j, b).wait()
        write_out(j, b)


_BB = 4096


def _tr_body(in_ref, out_ref):
    out_ref[...] = jnp.transpose(in_ref[...], (0, 2, 1))


_transpose = pl.pallas_call(
    _tr_body,
    out_shape=jax.ShapeDtypeStruct((FIELDS, WIDTH, BATCH), jnp.float32),
    grid=(FIELDS, BATCH // _BB),
    in_specs=[pl.BlockSpec((1, _BB, WIDTH), lambda f, i: (f, i, 0))],
    out_specs=pl.BlockSpec((1, WIDTH, _BB), lambda f, i: (f, 0, i)),
)

_PB = 8192
_NP = 507904


def _pack_body(a_ref, b_ref, out_ref):
    out_ref[:, :WIDTH] = jnp.transpose(a_ref[...], (1, 0))
    out_ref[:, WIDTH:] = jnp.transpose(b_ref[...], (1, 0))


_pack = pl.pallas_call(
    _pack_body,
    out_shape=jax.ShapeDtypeStruct((_NP, 2 * WIDTH), jnp.float32),
    grid=(_NP // _PB,),
    in_specs=[
        pl.BlockSpec((WIDTH, _PB), lambda i: (0, i)),
        pl.BlockSpec(
            (WIDTH, _PB),
            lambda i: (0, jnp.minimum(i + _NP // _PB, CODES // _PB)),
        ),
    ],
    out_specs=pl.BlockSpec((_PB, 2 * WIDTH), lambda i: (i, 0)),
)


def kernel(x, embed):
    v = jnp.where(jnp.isnan(x), jnp.float32(CODES), x).astype(jnp.int32)
    u = jnp.where(v < _NP, 2 * v, 2 * (v - _NP) + 1)
    u = u.T.reshape(B)
    tab = _pack(jnp.transpose(embed), jnp.transpose(embed))
    rows = _embed_gather(u, tab.reshape(2 * _NP, WIDTH))
    out = _transpose(rows.reshape(FIELDS, BATCH, WIDTH))
    return jnp.transpose(out, (2, 0, 1))


# speedup vs baseline: 5.6168x; 1.0533x over previous
import functools

import jax
import jax.numpy as jnp
from jax import lax
from jax.experimental import pallas as pl
from jax.experimental.pallas import tpu as pltpu
from jax.experimental.pallas import tpu_sc as plsc

CODES = 1000000
WIDTH = 64
BATCH = 16384
FIELDS = 26

NC = 2
NS = 16
NW = NC * NS
B = BATCH * FIELDS
BPW = B // NW
CHUNK = 128
NCHUNK = BPW // CHUNK
NBUF = 4
NGROUP = NCHUNK // NBUF

_mesh = plsc.VectorSubcoreMesh(
    core_axis_name="c", subcore_axis_name="s", num_cores=NC, num_subcores=NS
)


@functools.partial(
    pl.kernel,
    out_type=jax.ShapeDtypeStruct((B, WIDTH), jnp.float32),
    mesh=_mesh,
    compiler_params=pltpu.CompilerParams(use_tc_tiling_on_sc=False),
    scratch_types=[
        pltpu.VMEM((BPW,), jnp.int32),
        pltpu.VMEM((NBUF, CHUNK, WIDTH), jnp.float32),
        [pltpu.SemaphoreType.DMA] * NBUF,
    ],
)
def _embed_gather(idx_hbm, tab_hbm, out_hbm, idx_v, rows_v, gsems):
    wid = lax.axis_index("s") * NC + lax.axis_index("c")
    base = wid * BPW

    pltpu.sync_copy(idx_hbm.at[pl.ds(base, BPW)], idx_v)

    def gather(j, b):
        return pltpu.make_async_copy(
            tab_hbm.at[idx_v.at[pl.ds(j * CHUNK, CHUNK)]], rows_v.at[b], gsems[b]
        )

    def write_out(j, b):
        pltpu.sync_copy(rows_v.at[b], out_hbm.at[pl.ds(base + j * CHUNK, CHUNK)])

    for b in range(NBUF):
        gather(b, b).start()

    def group(gi, carry):
        for b in range(NBUF):
            j = gi * NBUF + b
            gather(j, b).wait()
            write_out(j, b)
            gather(j + NBUF, b).start()
        return carry

    lax.fori_loop(0, NGROUP - 1, group, 0)

    for b in range(NBUF):
        j = (NGROUP - 1) * NBUF + b
        gather(j, b).wait()
        write_out(j, b)


_BB = 8192


def _tr_body(in_ref, out_ref):
    out_ref[...] = jnp.transpose(in_ref[...], (0, 2, 1))


_transpose = pl.pallas_call(
    _tr_body,
    out_shape=jax.ShapeDtypeStruct((FIELDS, WIDTH, BATCH), jnp.float32),
    grid=(FIELDS, BATCH // _BB),
    in_specs=[pl.BlockSpec((1, _BB, WIDTH), lambda f, i: (f, i, 0))],
    out_specs=pl.BlockSpec((1, WIDTH, _BB), lambda f, i: (f, 0, i)),
)

_PB = 8192
_NP = 507904


def _pack_body(a_ref, b_ref, out_ref):
    out_ref[:, :WIDTH] = jnp.transpose(a_ref[...], (1, 0))
    out_ref[:, WIDTH:] = jnp.transpose(b_ref[...], (1, 0))


_pack = pl.pallas_call(
    _pack_body,
    out_shape=jax.ShapeDtypeStruct((_NP, 2 * WIDTH), jnp.float32),
    grid=(_NP // _PB,),
    in_specs=[
        pl.BlockSpec((WIDTH, _PB), lambda i: (0, i)),
        pl.BlockSpec(
            (WIDTH, _PB),
            lambda i: (0, jnp.minimum(i + _NP // _PB, CODES // _PB)),
        ),
    ],
    out_specs=pl.BlockSpec((_PB, 2 * WIDTH), lambda i: (i, 0)),
)


def kernel(x, embed):
    v = jnp.where(jnp.isnan(x), jnp.float32(CODES), x).astype(jnp.int32)
    u = jnp.where(v < _NP, 2 * v, 2 * (v - _NP) + 1)
    u = u.T.reshape(B)
    tab = _pack(jnp.transpose(embed), jnp.transpose(embed))
    rows = _embed_gather(u, tab.reshape(2 * _NP, WIDTH))
    out = _transpose(rows.reshape(FIELDS, BATCH, WIDTH))
    return jnp.transpose(out, (2, 0, 1))


# speedup vs baseline: 5.8291x; 1.0378x over previous
import functools

import jax
import jax.numpy as jnp
from jax import lax
from jax.experimental import pallas as pl
from jax.experimental.pallas import tpu as pltpu
from jax.experimental.pallas import tpu_sc as plsc

CODES = 1000000
WIDTH = 64
BATCH = 16384
FIELDS = 26

NC = 2
NS = 16
NW = NC * NS
B = BATCH * FIELDS
BPW = B // NW
CHUNK = 128
NCHUNK = BPW // CHUNK
NBUF = 4
NGROUP = NCHUNK // NBUF

_mesh = plsc.VectorSubcoreMesh(
    core_axis_name="c", subcore_axis_name="s", num_cores=NC, num_subcores=NS
)


@functools.partial(
    pl.kernel,
    out_type=jax.ShapeDtypeStruct((B, WIDTH), jnp.float32),
    mesh=_mesh,
    compiler_params=pltpu.CompilerParams(use_tc_tiling_on_sc=False),
    scratch_types=[
        pltpu.VMEM((BPW,), jnp.int32),
        pltpu.VMEM((NBUF, CHUNK, WIDTH), jnp.float32),
        [pltpu.SemaphoreType.DMA] * NBUF,
    ],
)
def _embed_gather(idx_hbm, tab_hbm, out_hbm, idx_v, rows_v, gsems):
    wid = lax.axis_index("s") * NC + lax.axis_index("c")
    base = wid * BPW

    pltpu.sync_copy(idx_hbm.at[pl.ds(base, BPW)], idx_v)

    def gather(j, b):
        return pltpu.make_async_copy(
            tab_hbm.at[idx_v.at[pl.ds(j * CHUNK, CHUNK)]], rows_v.at[b], gsems[b]
        )

    def write_out(j, b):
        pltpu.sync_copy(rows_v.at[b], out_hbm.at[pl.ds(base + j * CHUNK, CHUNK)])

    for b in range(NBUF):
        gather(b, b).start()

    def group(gi, carry):
        for b in range(NBUF):
            j = gi * NBUF + b
            gather(j, b).wait()
            write_out(j, b)
            gather(j + NBUF, b).start()
        return carry

    lax.fori_loop(0, NGROUP - 1, group, 0)

    for b in range(NBUF):
        j = (NGROUP - 1) * NBUF + b
        gather(j, b).wait()
        write_out(j, b)


_BB = 16384


def _tr_body(in_ref, out_ref):
    out_ref[...] = jnp.transpose(in_ref[...], (0, 2, 1))


_transpose = pl.pallas_call(
    _tr_body,
    out_shape=jax.ShapeDtypeStruct((FIELDS, WIDTH, BATCH), jnp.float32),
    grid=(FIELDS, BATCH // _BB),
    in_specs=[pl.BlockSpec((1, _BB, WIDTH), lambda f, i: (f, i, 0))],
    out_specs=pl.BlockSpec((1, WIDTH, _BB), lambda f, i: (f, 0, i)),
)

_PB = 16384
_NP = 507904


def _pack_body(a_ref, b_ref, out_ref):
    out_ref[:, :WIDTH] = jnp.transpose(a_ref[...], (1, 0))
    out_ref[:, WIDTH:] = jnp.transpose(b_ref[...], (1, 0))


_pack = pl.pallas_call(
    _pack_body,
    out_shape=jax.ShapeDtypeStruct((_NP, 2 * WIDTH), jnp.float32),
    grid=(_NP // _PB,),
    in_specs=[
        pl.BlockSpec((WIDTH, _PB), lambda i: (0, i)),
        pl.BlockSpec(
            (WIDTH, _PB),
            lambda i: (0, jnp.minimum(i + _NP // _PB, CODES // _PB)),
        ),
    ],
    out_specs=pl.BlockSpec((_PB, 2 * WIDTH), lambda i: (i, 0)),
)


def kernel(x, embed):
    v = jnp.where(jnp.isnan(x), jnp.float32(CODES), x).astype(jnp.int32)
    u = jnp.where(v < _NP, 2 * v, 2 * (v - _NP) + 1)
    u = u.T.reshape(B)
    tab = _pack(jnp.transpose(embed), jnp.transpose(embed))
    rows = _embed_gather(u, tab.reshape(2 * _NP, WIDTH))
    out = _transpose(rows.reshape(FIELDS, BATCH, WIDTH))
    return jnp.transpose(out, (2, 0, 1))
